# Initial kernel scaffold; baseline (speedup 1.0000x reference)
import jax
import jax.numpy as jnp
from jax.experimental import pallas as pl


def kernel(features, adj, coords, W1, b1, W2, b2):
    raise NotImplementedError("write your pallas kernel here")


# speedup vs baseline: 4.6874x; 4.6874x over previous
import functools

import jax
import jax.numpy as jnp
from jax import lax
from jax.experimental import pallas as pl
from jax.experimental.pallas import tpu as pltpu
from jax.experimental.pallas import tpu_sc as plsc

N = 4096
E = 65536
D = 256
NEIGHBOR = 16
GAMMA = 2.0
OMEGA = 0.8
EPS = 1e-08

NC = 2
NS = 16
NW = NC * NS
EPW = E // NW
CH = 128
NCH = EPW // CH

F32 = jnp.float32


_KTH_BR = 128


def _kth_body(cxr, cyr, cxc, cyc, out):
    xr = cxr[...]
    yr = cyr[...]
    xc = cxc[...]
    yc = cyc[...]
    dx = xr - xc
    dy = yr - yc
    d2 = dx * dx + dy * dy
    colid = lax.broadcasted_iota(jnp.int32, d2.shape, 1)
    inf = jnp.float32(jnp.inf)

    def body(_, d2):
        m = jnp.min(d2, axis=1, keepdims=True)
        pos = jnp.min(jnp.where(d2 == m, colid, N), axis=1, keepdims=True)
        return jnp.where(colid == pos, inf, d2)

    d2 = lax.fori_loop(0, NEIGHBOR, body, d2)
    out[...] = jnp.sqrt(jnp.min(d2, axis=1, keepdims=True))


def _kth_dist(cxr, cyr, cxc, cyc):
    br = _KTH_BR
    return pl.pallas_call(
        _kth_body,
        grid=(N // br,),
        in_specs=[
            pl.BlockSpec((br, 1), lambda i: (i, 0)),
            pl.BlockSpec((br, 1), lambda i: (i, 0)),
            pl.BlockSpec((1, N), lambda i: (0, 0)),
            pl.BlockSpec((1, N), lambda i: (0, 0)),
        ],
        out_specs=pl.BlockSpec((br, 1), lambda i: (i, 0)),
        out_shape=jax.ShapeDtypeStruct((N, 1), F32),
    )(cxr, cyr, cxc, cyc)


def _med_body(vr_ref, vc_ref, out_ref):
    vr = vr_ref[...]
    chunk = 512

    def body(j, cnt):
        blk = vc_ref[:, pl.ds(j * chunk, chunk)]
        le = (blk <= vr).astype(F32)
        return cnt + jnp.sum(le, axis=1, keepdims=True)

    cnt = lax.fori_loop(0, N // chunk, body, jnp.zeros((N, 1), F32))
    inf = jnp.float32(jnp.inf)
    half = N // 2
    a = jnp.min(jnp.where(cnt >= half, vr, inf))
    b = jnp.min(jnp.where(cnt >= half + 1, vr, inf))
    out_ref[...] = jnp.full((1, 1), 0.5 * (a + b), F32)


def _median(v_col, v_row):
    return pl.pallas_call(
        _med_body,
        out_shape=jax.ShapeDtypeStruct((1, 1), F32),
    )(v_col, v_row)


RPW = N // NW
MLPAD = 144
SCCH = 2048

_I16 = lambda: lax.broadcasted_iota(jnp.int32, (16,), 0)


def _sc_bucket_body(pk_hbm, ml_hbm, mlc_hbm, stage_v, ml_v, mlcv):
    cid = lax.axis_index("c")
    sid = lax.axis_index("s")
    wid = cid * NS + sid
    lo = wid * RPW

    i16 = _I16()
    folds = [(jnp.maximum(i16 - k, 0), i16 >= k) for k in (1, 2, 4, 8)]
    r1 = i16 + 1

    def scan_chunk(ci, off):
        pltpu.sync_copy(pk_hbm.at[pl.ds(ci * SCCH, SCCH)], stage_v)

        def scan_vec(k, off):
            pv = stage_v[pl.ds(k * 16, 16)]
            d = lax.shift_right_logical(pv, 12)
            mi = jnp.where(jnp.logical_and(d >= lo, d < lo + RPW),
                           1, 0).astype(jnp.int32)
            incl = mi
            for pidx, pm in folds:
                incl = incl + jnp.where(pm, jnp.take(incl, pidx, axis=0), 0)
            cnt = incl[15]
            b = jnp.zeros((16,), jnp.int32)
            for step in (8, 4, 2, 1):
                nb = b + step
                t = jnp.take(incl, nb - 1, axis=0)
                b = jnp.where(t < r1, nb, b)
            comp = jnp.take(pv, jnp.minimum(b, 15), axis=0)
            ml_v[pl.ds(off, 16)] = comp
            return off + cnt

        return lax.fori_loop(0, SCCH // 16, scan_vec, off)

    off = lax.fori_loop(0, E // SCCH, scan_chunk, jnp.int32(0))

    neg1 = jnp.full((16,), -1, jnp.int32)
    for k in range(8):
        ml_v[pl.ds(off + k * 16, 16)] = neg1
    off2 = ((off + 127) >> 7) << 7

    mlcv[...] = jnp.full((16,), off2, jnp.int32)
    pltpu.sync_copy(ml_v, ml_hbm.at[wid])
    pltpu.sync_copy(mlcv, mlc_hbm.at[wid])


def _sc_bucket(packed):
    mesh = plsc.VectorSubcoreMesh(core_axis_name="c", subcore_axis_name="s",
                                  num_cores=NC, num_subcores=NS)
    fn = pl.kernel(
        _sc_bucket_body,
        out_type=[
            jax.ShapeDtypeStruct((NW, E + MLPAD), jnp.int32),
            jax.ShapeDtypeStruct((NW, 16), jnp.int32),
        ],
        mesh=mesh,
        scratch_types=[
            pltpu.VMEM((SCCH,), jnp.int32),
            pltpu.VMEM((E + MLPAD,), jnp.int32),
            pltpu.VMEM((16,), jnp.int32),
        ],
    )
    return fn(packed)


def _sc_seg_body(ys_hbm, ml_hbm, mlc_hbm, zf_hbm, out_hbm,
                 pe_v, sidx_v, rows_v, acc_v, mlcv, sem):
    cid = lax.axis_index("c")
    sid = lax.axis_index("s")
    wid = cid * NS + sid
    lo = wid * RPW

    pltpu.sync_copy(mlc_hbm.at[wid], mlcv)
    v0 = mlcv[pl.ds(0, 16)]
    nch = lax.shift_right_logical(v0[0], 7)

    pltpu.sync_copy(zf_hbm.at[pl.ds(0, RPW)], acc_v)

    def chunk(ci, carry):
        pltpu.sync_copy(ml_hbm.at[wid, pl.ds(ci * CH, CH)],
                        pe_v.at[pl.ds(0, CH)])
        for k in range(CH // 16):
            pv = pe_v[pl.ds(k * 16, 16)]
            sidx_v[pl.ds(k * 16, 16)] = jnp.where(
                pv < 0, 0, jnp.bitwise_and(pv, 4095))
        pltpu.async_copy(ys_hbm.at[sidx_v], rows_v, sem).wait()

        def edge(_, e):
            v = pe_v[pl.ds(e, 16)]
            pv = v[0]

            @pl.when(pv >= 0)
            def _():
                dloc = lax.shift_right_logical(pv, 12) - lo
                for cc in range(D // 16):
                    x = rows_v[e, pl.ds(cc * 16, 16)]
                    plsc.addupdate(acc_v.at[dloc, pl.ds(cc * 16, 16)], x)

            return e + 1

        lax.fori_loop(0, CH, edge, jnp.int32(0))
        return carry

    lax.fori_loop(0, nch, chunk, 0)
    pltpu.sync_copy(acc_v, out_hbm.at[pl.ds(lo, RPW)])


def _sc_segment_sum(ys, ml, mlc, zeros_nd):
    mesh = plsc.VectorSubcoreMesh(core_axis_name="c", subcore_axis_name="s",
                                  num_cores=NC, num_subcores=NS)
    fn = pl.kernel(
        _sc_seg_body,
        out_type=jax.ShapeDtypeStruct((N, D), F32),
        mesh=mesh,
        scratch_types=[
            pltpu.VMEM((CH + 16,), jnp.int32),
            pltpu.VMEM((CH,), jnp.int32),
            pltpu.VMEM((CH, D), F32),
            pltpu.VMEM((RPW, D), F32),
            pltpu.VMEM((16,), jnp.int32),
            pltpu.SemaphoreType.DMA,
        ],
    )
    return fn(ys, ml, mlc, zeros_nd)


def _pack_body(s_ref, d_ref, out_ref):
    out_ref[...] = d_ref[...] * 4096 + s_ref[...]


def _pack(src2d, dst2d):
    return pl.pallas_call(
        _pack_body,
        out_shape=jax.ShapeDtypeStruct((E // 128, 128), jnp.int32),
    )(src2d, dst2d)


_MM_BM = 512


def _dinv_block(deg_ref):
    deg = deg_ref[...] + 1.0
    return 1.0 / jnp.sqrt(deg)


def _mm_scale_body(x_ref, w_ref, deg_ref, out_ref):
    dinv = _dinv_block(deg_ref)
    xw = jnp.dot(x_ref[...], w_ref[...], preferred_element_type=F32)
    out_ref[...] = xw * dinv


def _mm_scale(x, w, deg):
    bm = _MM_BM
    return pl.pallas_call(
        _mm_scale_body,
        grid=(N // bm,),
        in_specs=[
            pl.BlockSpec((bm, D), lambda i: (i, 0)),
            pl.BlockSpec((D, D), lambda i: (0, 0)),
            pl.BlockSpec((bm, 1), lambda i: (i, 0)),
        ],
        out_specs=pl.BlockSpec((bm, D), lambda i: (i, 0)),
        out_shape=jax.ShapeDtypeStruct((N, D), F32),
    )(x, w, deg)


def _layer2_body(a_ref, ys1_ref, b1_ref, w_ref, deg_ref, out_ref):
    dinv = _dinv_block(deg_ref)
    h1 = (a_ref[...] + ys1_ref[...]) * dinv + b1_ref[...]
    h1 = jnp.maximum(h1, 0.0)
    out_ref[...] = jnp.dot(h1, w_ref[...], preferred_element_type=F32) * dinv


def _layer2(a, ys1, b1, w2, deg):
    bm = _MM_BM
    return pl.pallas_call(
        _layer2_body,
        grid=(N // bm,),
        in_specs=[
            pl.BlockSpec((bm, D), lambda i: (i, 0)),
            pl.BlockSpec((bm, D), lambda i: (i, 0)),
            pl.BlockSpec((1, D), lambda i: (0, 0)),
            pl.BlockSpec((D, D), lambda i: (0, 0)),
            pl.BlockSpec((bm, 1), lambda i: (i, 0)),
        ],
        out_specs=pl.BlockSpec((bm, D), lambda i: (i, 0)),
        out_shape=jax.ShapeDtypeStruct((N, D), F32),
    )(a, ys1, b1, w2, deg)


def _final_h_body(a_ref, ys2_ref, b2_ref, deg_ref, h_ref, hn_ref):
    dinv = _dinv_block(deg_ref)
    h = (a_ref[...] + ys2_ref[...]) * dinv + b2_ref[...]
    h_ref[...] = h
    hn_ref[...] = jnp.sqrt(jnp.sum(h * h, axis=1, keepdims=True))


def _final_h(a, ys2, b2, deg):
    bm = _MM_BM
    return pl.pallas_call(
        _final_h_body,
        grid=(N // bm,),
        in_specs=[
            pl.BlockSpec((bm, D), lambda i: (i, 0)),
            pl.BlockSpec((bm, D), lambda i: (i, 0)),
            pl.BlockSpec((1, D), lambda i: (0, 0)),
            pl.BlockSpec((bm, 1), lambda i: (i, 0)),
        ],
        out_specs=[
            pl.BlockSpec((bm, D), lambda i: (i, 0)),
            pl.BlockSpec((bm, 1), lambda i: (i, 0)),
        ],
        out_shape=[
            jax.ShapeDtypeStruct((N, D), F32),
            jax.ShapeDtypeStruct((N, 1), F32),
        ],
    )(a, ys2, b2, deg)


_SIM_BM = 512
_SIM_BN = 512


def _sim_body(hi_ref, hj_ref, hni_ref, hnj_ref, cxr_ref, cyr_ref,
              cxc_ref, cyc_ref, dcut_ref, s_ref, deg_ref, acc):
    j = pl.program_id(1)
    g = lax.dot_general(hi_ref[...], hj_ref[...],
                        (((1,), (1,)), ((), ())),
                        preferred_element_type=F32)
    s1 = g / (hni_ref[...] * hnj_ref[...] + EPS)
    s1 = jnp.maximum(s1, 0.0)
    dx = cxr_ref[...] - cxc_ref[...]
    dy = cyr_ref[...] - cyc_ref[...]
    d = jnp.sqrt(dx * dx + dy * dy)
    dc = dcut_ref[0, 0]
    mask = jnp.logical_and(d > 0.0, d <= dc)
    t = d / dc
    sd = jnp.where(mask, 1.0 / jnp.exp(GAMMA * (t * t)), 0.0)
    s = jnp.where(mask, OMEGA * s1 + (1.0 - OMEGA) * sd, 0.0)
    s_ref[...] = s
    rs = jnp.sum(s, axis=1, keepdims=True)

    @pl.when(j == 0)
    def _():
        acc[...] = rs

    @pl.when(j > 0)
    def _():
        acc[...] = acc[...] + rs

    deg_ref[...] = acc[...]


def _sim(h, hn, hn_t, cxr, cyr, cxc, cyc, dcut):
    bm, bn = _SIM_BM, _SIM_BN
    return pl.pallas_call(
        _sim_body,
        grid=(N // bm, N // bn),
        in_specs=[
            pl.BlockSpec((bm, D), lambda i, j: (i, 0)),
            pl.BlockSpec((bn, D), lambda i, j: (j, 0)),
            pl.BlockSpec((bm, 1), lambda i, j: (i, 0)),
            pl.BlockSpec((1, bn), lambda i, j: (0, j)),
            pl.BlockSpec((bm, 1), lambda i, j: (i, 0)),
            pl.BlockSpec((bm, 1), lambda i, j: (i, 0)),
            pl.BlockSpec((1, bn), lambda i, j: (0, j)),
            pl.BlockSpec((1, bn), lambda i, j: (0, j)),
            pl.BlockSpec((1, 1), lambda i, j: (0, 0)),
        ],
        out_specs=[
            pl.BlockSpec((bm, bn), lambda i, j: (i, j)),
            pl.BlockSpec((bm, 1), lambda i, j: (i, 0)),
        ],
        out_shape=[
            jax.ShapeDtypeStruct((N, N), F32),
            jax.ShapeDtypeStruct((N, 1), F32),
        ],
        scratch_shapes=[pltpu.VMEM((bm, 1), F32)],
    )(h, h, hn, hn_t, cxr, cyr, cxc, cyc, dcut)


def _norm_body(s_ref, dr_ref, dc_ref, out_ref):
    dr = dr_ref[...]
    dc = dc_ref[...]
    dinv_r = jnp.where(dr > 0.0, 1.0 / jnp.sqrt(dr), 0.0)
    dinv_c = jnp.where(dc > 0.0, 1.0 / jnp.sqrt(dc), 0.0)
    out_ref[...] = s_ref[...] * dinv_r * dinv_c


def _norm(s, deg_row, deg_col):
    bm, bn = _SIM_BM, _SIM_BN
    return pl.pallas_call(
        _norm_body,
        grid=(N // bm, N // bn),
        in_specs=[
            pl.BlockSpec((bm, bn), lambda i, j: (i, j)),
            pl.BlockSpec((bm, 1), lambda i, j: (i, 0)),
            pl.BlockSpec((1, bn), lambda i, j: (0, j)),
        ],
        out_specs=pl.BlockSpec((bm, bn), lambda i, j: (i, j)),
        out_shape=jax.ShapeDtypeStruct((N, N), F32),
    )(s, deg_row, deg_col)


def kernel(features, adj, coords, W1, b1, W2, b2):
    src = adj[:, 0].astype(jnp.int32)
    dst = adj[:, 1].astype(jnp.int32)
    cx = coords[:, 0].astype(F32)
    cy = coords[:, 1].astype(F32)
    cxr = cx.reshape(N, 1)
    cyr = cy.reshape(N, 1)
    cxc = cx.reshape(1, N)
    cyc = cy.reshape(1, N)

    kth = _kth_dist(cxr, cyr, cxc, cyc)
    dcut = _median(kth, kth.reshape(1, N))

    packed = _pack(src.reshape(E // 128, 128), dst.reshape(E // 128, 128))
    ml, mlc = _sc_bucket(packed.reshape(E))
    zeros_nd = jnp.zeros((N, D), F32)
    ones_nd = jnp.ones((N, D), F32)
    deg = _sc_segment_sum(ones_nd, ml, mlc, zeros_nd)[:, :1]

    ys1 = _mm_scale(features, W1, deg)
    a1 = _sc_segment_sum(ys1, ml, mlc, zeros_nd)
    ys2 = _layer2(a1, ys1, b1.reshape(1, D), W2, deg)
    a2 = _sc_segment_sum(ys2, ml, mlc, zeros_nd)
    h, hn = _final_h(a2, ys2, b2.reshape(1, D), deg)

    s, deg2 = _sim(h, hn, hn.reshape(1, N), cxr, cyr, cxc, cyc, dcut)
    s_norm = _norm(s, deg2, deg2.reshape(1, N))
    return (s_norm, s)


# speedup vs baseline: 5.7128x; 1.2188x over previous
import functools

import jax
import jax.numpy as jnp
from jax import lax
from jax.experimental import pallas as pl
from jax.experimental.pallas import tpu as pltpu
from jax.experimental.pallas import tpu_sc as plsc

N = 4096
E = 65536
D = 256
NEIGHBOR = 16
GAMMA = 2.0
OMEGA = 0.8
EPS = 1e-08

NC = 2
NS = 16
NW = NC * NS
EPW = E // NW
CH = 128
NCH = EPW // CH

F32 = jnp.float32


_KTH_BR = 128


def _kth_body(cxr, cyr, cxc, cyc, out):
    xr = cxr[...]
    yr = cyr[...]
    xc = cxc[...]
    yc = cyc[...]
    dx = xr - xc
    dy = yr - yc
    d2 = dx * dx + dy * dy
    colid = lax.broadcasted_iota(jnp.int32, d2.shape, 1)
    inf = jnp.float32(jnp.inf)

    def body(_, d2):
        m = jnp.min(d2, axis=1, keepdims=True)
        pos = jnp.min(jnp.where(d2 == m, colid, N), axis=1, keepdims=True)
        return jnp.where(colid == pos, inf, d2)

    d2 = lax.fori_loop(0, NEIGHBOR, body, d2)
    out[...] = jnp.sqrt(jnp.min(d2, axis=1, keepdims=True))


def _kth_dist(cxr, cyr, cxc, cyc):
    br = _KTH_BR
    return pl.pallas_call(
        _kth_body,
        grid=(N // br,),
        in_specs=[
            pl.BlockSpec((br, 1), lambda i: (i, 0)),
            pl.BlockSpec((br, 1), lambda i: (i, 0)),
            pl.BlockSpec((1, N), lambda i: (0, 0)),
            pl.BlockSpec((1, N), lambda i: (0, 0)),
        ],
        out_specs=pl.BlockSpec((br, 1), lambda i: (i, 0)),
        out_shape=jax.ShapeDtypeStruct((N, 1), F32),
    )(cxr, cyr, cxc, cyc)


def _med_body(vr_ref, vc_ref, out_ref):
    vr = vr_ref[...]
    chunk = 512

    def body(j, cnt):
        blk = vc_ref[:, pl.ds(j * chunk, chunk)]
        le = (blk <= vr).astype(F32)
        return cnt + jnp.sum(le, axis=1, keepdims=True)

    cnt = lax.fori_loop(0, N // chunk, body, jnp.zeros((N, 1), F32))
    inf = jnp.float32(jnp.inf)
    half = N // 2
    a = jnp.min(jnp.where(cnt >= half, vr, inf))
    b = jnp.min(jnp.where(cnt >= half + 1, vr, inf))
    out_ref[...] = jnp.full((1, 1), 0.5 * (a + b), F32)


def _median(v_col, v_row):
    return pl.pallas_call(
        _med_body,
        out_shape=jax.ShapeDtypeStruct((1, 1), F32),
    )(v_col, v_row)


RPW = N // NW
MLPAD = 144
SCCH = 2048

_I16 = lambda: lax.broadcasted_iota(jnp.int32, (16,), 0)


def _sc_bucket_body(pk_hbm, zf_hbm, ml_hbm, deg_hbm, mlc_hbm, stage_v, ml_v,
                    deg2f_v, mlcv):
    cid = lax.axis_index("c")
    sid = lax.axis_index("s")
    wid = cid * NS + sid
    lo = wid * RPW

    i16 = _I16()
    folds = [(jnp.maximum(i16 - k, 0), i16 >= k) for k in (1, 2, 4, 8)]
    r1 = i16 + 1

    def scan_chunk(ci, off):
        pltpu.sync_copy(pk_hbm.at[pl.ds(ci * SCCH, SCCH)], stage_v)

        def scan_vec(k, off):
            pv = stage_v[pl.ds(k * 16, 16)]
            d = lax.shift_right_logical(pv, 12)
            mi = jnp.where(jnp.logical_and(d >= lo, d < lo + RPW),
                           1, 0).astype(jnp.int32)
            incl = mi
            for pidx, pm in folds:
                incl = incl + jnp.where(pm, jnp.take(incl, pidx, axis=0), 0)
            cnt = incl[15]
            b = jnp.zeros((16,), jnp.int32)
            for step in (8, 4, 2, 1):
                nb = b + step
                t = jnp.take(incl, nb - 1, axis=0)
                b = jnp.where(t < r1, nb, b)
            comp = jnp.take(pv, jnp.minimum(b, 15), axis=0)
            ml_v[pl.ds(off, 16)] = comp
            return off + cnt

        return lax.fori_loop(0, SCCH // 16, scan_vec, off)

    off = lax.fori_loop(0, E // SCCH, scan_chunk, jnp.int32(0))

    neg1 = jnp.full((16,), -1, jnp.int32)
    for k in range(8):
        ml_v[pl.ds(off + k * 16, 16)] = neg1
    off2 = ((off + 127) >> 7) << 7

    pltpu.sync_copy(zf_hbm.at[pl.ds(0, RPW * D)], deg2f_v)
    one0 = jnp.where(i16 == 0, 1.0, 0.0).astype(F32)
    nv = (off + 15) >> 4

    def cvec(kv, e):
        pv16 = ml_v[pl.ds(kv * 16, 16)]
        for j in range(16):
            pvj = pv16[j]

            @pl.when(pvj >= 0)
            def _(pvj=pvj, j=j):
                dloc = lax.shift_right_logical(pvj, 12) - lo
                addr = (dloc << 8) + j * 16
                plsc.addupdate(deg2f_v.at[pl.ds(addr, 16)], one0)

        return e + 16

    lax.fori_loop(0, nv, cvec, jnp.int32(0))
    pltpu.sync_copy(deg2f_v, deg_hbm.at[pl.ds(lo * D, RPW * D)])

    mlcv[...] = jnp.full((16,), off2, jnp.int32)
    pltpu.sync_copy(ml_v, ml_hbm.at[wid])
    pltpu.sync_copy(mlcv, mlc_hbm.at[wid])


def _sc_bucket(packed, zeros_flat):
    mesh = plsc.VectorSubcoreMesh(core_axis_name="c", subcore_axis_name="s",
                                  num_cores=NC, num_subcores=NS)
    fn = pl.kernel(
        _sc_bucket_body,
        out_type=[
            jax.ShapeDtypeStruct((NW, E + MLPAD), jnp.int32),
            jax.ShapeDtypeStruct((N * D,), F32),
            jax.ShapeDtypeStruct((NW, 16), jnp.int32),
        ],
        mesh=mesh,
        scratch_types=[
            pltpu.VMEM((SCCH,), jnp.int32),
            pltpu.VMEM((E + MLPAD,), jnp.int32),
            pltpu.VMEM((RPW * D,), F32),
            pltpu.VMEM((16,), jnp.int32),
        ],
    )
    return fn(packed, zeros_flat)


def _sc_seg_body(ys_hbm, ml_hbm, mlc_hbm, zf_hbm, out_hbm,
                 pe_v, sidx_v, rows_v, acc_v, mlcv, sem):
    cid = lax.axis_index("c")
    sid = lax.axis_index("s")
    wid = cid * NS + sid
    lo = wid * RPW

    pltpu.sync_copy(mlc_hbm.at[wid], mlcv)
    v0 = mlcv[pl.ds(0, 16)]
    nch = lax.shift_right_logical(v0[0], 7)

    pltpu.sync_copy(zf_hbm.at[pl.ds(0, RPW)], acc_v)

    def chunk(ci, carry):
        pltpu.sync_copy(ml_hbm.at[wid, pl.ds(ci * CH, CH)],
                        pe_v.at[pl.ds(0, CH)])
        for k in range(CH // 16):
            pv = pe_v[pl.ds(k * 16, 16)]
            sidx_v[pl.ds(k * 16, 16)] = jnp.where(
                pv < 0, 0, jnp.bitwise_and(pv, 4095))
        pltpu.async_copy(ys_hbm.at[sidx_v], rows_v, sem).wait()

        for k in range(CH // 16):
            pv16 = pe_v[pl.ds(k * 16, 16)]
            for j in range(16):
                pvj = pv16[j]

                @pl.when(pvj >= 0)
                def _(pvj=pvj, e=k * 16 + j):
                    dloc = lax.shift_right_logical(pvj, 12) - lo
                    for cc in range(D // 16):
                        x = rows_v[e, pl.ds(cc * 16, 16)]
                        plsc.addupdate(acc_v.at[dloc, pl.ds(cc * 16, 16)], x)

        return carry

    lax.fori_loop(0, nch, chunk, 0)
    pltpu.sync_copy(acc_v, out_hbm.at[pl.ds(lo, RPW)])


def _sc_segment_sum(ys, ml, mlc, zeros_nd):
    mesh = plsc.VectorSubcoreMesh(core_axis_name="c", subcore_axis_name="s",
                                  num_cores=NC, num_subcores=NS)
    fn = pl.kernel(
        _sc_seg_body,
        out_type=jax.ShapeDtypeStruct((N, D), F32),
        mesh=mesh,
        scratch_types=[
            pltpu.VMEM((CH + 16,), jnp.int32),
            pltpu.VMEM((CH,), jnp.int32),
            pltpu.VMEM((CH, D), F32),
            pltpu.VMEM((RPW, D), F32),
            pltpu.VMEM((16,), jnp.int32),
            pltpu.SemaphoreType.DMA,
        ],
    )
    return fn(ys, ml, mlc, zeros_nd)


def _pack_body(s_ref, d_ref, out_ref):
    out_ref[...] = d_ref[...] * 4096 + s_ref[...]


def _pack(src2d, dst2d):
    return pl.pallas_call(
        _pack_body,
        out_shape=jax.ShapeDtypeStruct((E // 128, 128), jnp.int32),
    )(src2d, dst2d)


_MM_BM = 512


def _dinv_block(deg_ref):
    deg = jnp.sum(deg_ref[...], axis=1, keepdims=True) + 1.0
    return 1.0 / jnp.sqrt(deg)


def _mm_scale_body(x_ref, w_ref, deg_ref, out_ref):
    dinv = _dinv_block(deg_ref)
    xw = jnp.dot(x_ref[...], w_ref[...], preferred_element_type=F32)
    out_ref[...] = xw * dinv


def _mm_scale(x, w, deg):
    bm = _MM_BM
    return pl.pallas_call(
        _mm_scale_body,
        grid=(N // bm,),
        in_specs=[
            pl.BlockSpec((bm, D), lambda i: (i, 0)),
            pl.BlockSpec((D, D), lambda i: (0, 0)),
            pl.BlockSpec((bm, D), lambda i: (i, 0)),
        ],
        out_specs=pl.BlockSpec((bm, D), lambda i: (i, 0)),
        out_shape=jax.ShapeDtypeStruct((N, D), F32),
    )(x, w, deg)


def _layer2_body(a_ref, ys1_ref, b1_ref, w_ref, deg_ref, out_ref):
    dinv = _dinv_block(deg_ref)
    h1 = (a_ref[...] + ys1_ref[...]) * dinv + b1_ref[...]
    h1 = jnp.maximum(h1, 0.0)
    out_ref[...] = jnp.dot(h1, w_ref[...], preferred_element_type=F32) * dinv


def _layer2(a, ys1, b1, w2, deg):
    bm = _MM_BM
    return pl.pallas_call(
        _layer2_body,
        grid=(N // bm,),
        in_specs=[
            pl.BlockSpec((bm, D), lambda i: (i, 0)),
            pl.BlockSpec((bm, D), lambda i: (i, 0)),
            pl.BlockSpec((1, D), lambda i: (0, 0)),
            pl.BlockSpec((D, D), lambda i: (0, 0)),
            pl.BlockSpec((bm, D), lambda i: (i, 0)),
        ],
        out_specs=pl.BlockSpec((bm, D), lambda i: (i, 0)),
        out_shape=jax.ShapeDtypeStruct((N, D), F32),
    )(a, ys1, b1, w2, deg)


def _final_h_body(a_ref, ys2_ref, b2_ref, deg_ref, h_ref, hn_ref):
    dinv = _dinv_block(deg_ref)
    h = (a_ref[...] + ys2_ref[...]) * dinv + b2_ref[...]
    h_ref[...] = h
    hn_ref[...] = jnp.sqrt(jnp.sum(h * h, axis=1, keepdims=True))


def _final_h(a, ys2, b2, deg):
    bm = _MM_BM
    return pl.pallas_call(
        _final_h_body,
        grid=(N // bm,),
        in_specs=[
            pl.BlockSpec((bm, D), lambda i: (i, 0)),
            pl.BlockSpec((bm, D), lambda i: (i, 0)),
            pl.BlockSpec((1, D), lambda i: (0, 0)),
            pl.BlockSpec((bm, D), lambda i: (i, 0)),
        ],
        out_specs=[
            pl.BlockSpec((bm, D), lambda i: (i, 0)),
            pl.BlockSpec((bm, 1), lambda i: (i, 0)),
        ],
        out_shape=[
            jax.ShapeDtypeStruct((N, D), F32),
            jax.ShapeDtypeStruct((N, 1), F32),
        ],
    )(a, ys2, b2, deg)


_SIM_BM = 512
_SIM_BN = 512


def _sim_body(hi_ref, hj_ref, hni_ref, hnj_ref, cxr_ref, cyr_ref,
              cxc_ref, cyc_ref, dcut_ref, s_ref, deg_ref, acc):
    j = pl.program_id(1)
    g = lax.dot_general(hi_ref[...], hj_ref[...],
                        (((1,), (1,)), ((), ())),
                        preferred_element_type=F32)
    s1 = g / (hni_ref[...] * hnj_ref[...] + EPS)
    s1 = jnp.maximum(s1, 0.0)
    dx = cxr_ref[...] - cxc_ref[...]
    dy = cyr_ref[...] - cyc_ref[...]
    d = jnp.sqrt(dx * dx + dy * dy)
    dc = dcut_ref[0, 0]
    mask = jnp.logical_and(d > 0.0, d <= dc)
    t = d / dc
    sd = jnp.where(mask, 1.0 / jnp.exp(GAMMA * (t * t)), 0.0)
    s = jnp.where(mask, OMEGA * s1 + (1.0 - OMEGA) * sd, 0.0)
    s_ref[...] = s
    rs = jnp.sum(s, axis=1, keepdims=True)

    @pl.when(j == 0)
    def _():
        acc[...] = rs

    @pl.when(j > 0)
    def _():
        acc[...] = acc[...] + rs

    deg_ref[...] = acc[...]


def _sim(h, hn, hn_t, cxr, cyr, cxc, cyc, dcut):
    bm, bn = _SIM_BM, _SIM_BN
    return pl.pallas_call(
        _sim_body,
        grid=(N // bm, N // bn),
        in_specs=[
            pl.BlockSpec((bm, D), lambda i, j: (i, 0)),
            pl.BlockSpec((bn, D), lambda i, j: (j, 0)),
            pl.BlockSpec((bm, 1), lambda i, j: (i, 0)),
            pl.BlockSpec((1, bn), lambda i, j: (0, j)),
            pl.BlockSpec((bm, 1), lambda i, j: (i, 0)),
            pl.BlockSpec((bm, 1), lambda i, j: (i, 0)),
            pl.BlockSpec((1, bn), lambda i, j: (0, j)),
            pl.BlockSpec((1, bn), lambda i, j: (0, j)),
            pl.BlockSpec((1, 1), lambda i, j: (0, 0)),
        ],
        out_specs=[
            pl.BlockSpec((bm, bn), lambda i, j: (i, j)),
            pl.BlockSpec((bm, 1), lambda i, j: (i, 0)),
        ],
        out_shape=[
            jax.ShapeDtypeStruct((N, N), F32),
            jax.ShapeDtypeStruct((N, 1), F32),
        ],
        scratch_shapes=[pltpu.VMEM((bm, 1), F32)],
    )(h, h, hn, hn_t, cxr, cyr, cxc, cyc, dcut)


def _norm_body(s_ref, dr_ref, dc_ref, out_ref):
    dr = dr_ref[...]
    dc = dc_ref[...]
    dinv_r = jnp.where(dr > 0.0, 1.0 / jnp.sqrt(dr), 0.0)
    dinv_c = jnp.where(dc > 0.0, 1.0 / jnp.sqrt(dc), 0.0)
    out_ref[...] = s_ref[...] * dinv_r * dinv_c


def _norm(s, deg_row, deg_col):
    bm, bn = _SIM_BM, _SIM_BN
    return pl.pallas_call(
        _norm_body,
        grid=(N // bm, N // bn),
        in_specs=[
            pl.BlockSpec((bm, bn), lambda i, j: (i, j)),
            pl.BlockSpec((bm, 1), lambda i, j: (i, 0)),
            pl.BlockSpec((1, bn), lambda i, j: (0, j)),
        ],
        out_specs=pl.BlockSpec((bm, bn), lambda i, j: (i, j)),
        out_shape=jax.ShapeDtypeStruct((N, N), F32),
    )(s, deg_row, deg_col)


def kernel(features, adj, coords, W1, b1, W2, b2):
    src = adj[:, 0].astype(jnp.int32)
    dst = adj[:, 1].astype(jnp.int32)
    cx = coords[:, 0].astype(F32)
    cy = coords[:, 1].astype(F32)
    cxr = cx.reshape(N, 1)
    cyr = cy.reshape(N, 1)
    cxc = cx.reshape(1, N)
    cyc = cy.reshape(1, N)

    kth = _kth_dist(cxr, cyr, cxc, cyc)
    dcut = _median(kth, kth.reshape(1, N))

    zeros_nd = jnp.zeros((N, D), F32)
    packed = _pack(src.reshape(E // 128, 128), dst.reshape(E // 128, 128))
    ml, degf, mlc = _sc_bucket(packed.reshape(E), zeros_nd.reshape(N * D))
    deg = degf.reshape(N, D)

    ys1 = _mm_scale(features, W1, deg)
    a1 = _sc_segment_sum(ys1, ml, mlc, zeros_nd)
    ys2 = _layer2(a1, ys1, b1.reshape(1, D), W2, deg)
    a2 = _sc_segment_sum(ys2, ml, mlc, zeros_nd)
    h, hn = _final_h(a2, ys2, b2.reshape(1, D), deg)

    s, deg2 = _sim(h, hn, hn.reshape(1, N), cxr, cyr, cxc, cyc, dcut)
    s_norm = _norm(s, deg2, deg2.reshape(1, N))
    return (s_norm, s)
